# baseline (speedup 1.0000x reference)
import functools

import jax
import jax.numpy as jnp
from jax import lax
from jax.experimental import pallas as pl
from jax.experimental.pallas import tpu as pltpu
from jax.experimental.pallas import tpu_sc as plsc

H = 512
W = 512
C = 8
NPIX = H * W
HALF = 12.8
SCALE = 256.0 / HALF
SHIFT = 255.5
TROWS = H * W

_info = plsc.get_sparse_core_info()
NC, NS = _info.num_cores, _info.num_subcores
NW = NC * NS
ROWS_PER_W = H // NW
NVEC = W // 16
NDMA = W // 128
BY = H // NS

_f32 = jnp.float32
_i32 = jnp.int32


@functools.partial(
    pl.kernel,
    mesh=plsc.VectorSubcoreMesh(core_axis_name="c", subcore_axis_name="s"),
    out_type=(
        jax.ShapeDtypeStruct((1, C, H, W), _f32),
        jax.ShapeDtypeStruct((NC * TROWS, 16), _f32),
    ),
    compiler_params=pltpu.CompilerParams(
        needs_layout_passes=False, use_tc_tiling_on_sc=False),
    scratch_types=[
        pltpu.VMEM((2, C * W + 8), _f32),
        pltpu.VMEM((2, W, 16), _f32),
        pltpu.VMEM((ROWS_PER_W * W,), _f32),
        pltpu.VMEM((ROWS_PER_W * W,), _f32),
        pltpu.VMEM((2, W), _i32),
        pltpu.VMEM((2, W), _i32),
        pltpu.VMEM((2, W), _f32),
        pltpu.VMEM((2, W), _f32),
        pltpu.VMEM((2, W), _f32),
        pltpu.VMEM((2, W), _f32),
        pltpu.VMEM((2, W, 16), _f32),
        pltpu.VMEM((2, W, 16), _f32),
        pltpu.VMEM((C, 2, W), _f32),
        pltpu.SemaphoreType.DMA,
        pltpu.SemaphoreType.DMA,
        pltpu.SemaphoreType.DMA,
        pltpu.SemaphoreType.DMA,
        pltpu.SemaphoreType.DMA,
        pltpu.SemaphoreType.DMA,
        pltpu.SemaphoreType.DMA,
        pltpu.SemaphoreType.DMA,
    ],
)
def _lens_sc(img_hbm, alpha_hbm, out_hbm, table_hbm,
             inb_v, ebo_v, ax_v, ay_v, ri0_v, ri1_v,
             w00_v, w01_v, w10_v, w11_v, g0_v, g1_v, outr_v,
             sem_a, sem_i0, sem_i1, sem_t0, sem_t1, sem_ga, sem_gb, sem_o):
    sc = lax.axis_index("c")
    ss = lax.axis_index("s")
    wid = ss * NC + sc
    base_row = wid * ROWS_PER_W
    lane = lax.iota(_i32, 16)
    ch_pat = lane & 7
    px_pat = lane >> 3
    tbase = sc * TROWS
    sem_i = (sem_i0, sem_i1)
    sem_t = (sem_t0, sem_t1)

    a_cps = [
        pltpu.async_copy(
            alpha_hbm.at[pl.ds(p * NPIX + base_row * W, ROWS_PER_W * W)],
            av, sem_a)
        for p, av in ((0, ax_v), (1, ay_v))
    ]

    y_base = ss * BY

    def fire_build(g):
        return [
            pltpu.async_copy(img_hbm.at[ch, pl.ds((y_base + g) * W, W)],
                             inb_v.at[g & 1, pl.ds(ch * W, W)], sem_i[g & 1])
            for ch in range(C)
        ]

    def build_group(g):
        pb = g & 1
        inb_p = inb_v.at[pb]
        ebo_p = ebo_v.at[pb]

        def make_body(half):
            def build_row(t, addr):
                ebo_p[half * 256 + t, :] = plsc.load_gather(inb_p, [addr])
                return addr + 2
            return build_row

        lax.fori_loop(0, W // 2, make_body(0),
                      ch_pat * W + px_pat, unroll=4)
        lax.fori_loop(0, W // 2, make_body(1),
                      ch_pat * W + px_pat + 1, unroll=4)
        return pltpu.async_copy(
            ebo_p,
            table_hbm.at[pl.ds(tbase + (y_base + g) * W, W)], sem_t[pb])

    pend_b = fire_build(0)
    pend_t = [None, None]
    for g in range(BY):
        nxt = fire_build(g + 1) if g + 1 < BY else None
        for cp in pend_b:
            cp.wait()
        if pend_t[g & 1] is not None:
            pend_t[g & 1].wait()
        pend_t[g & 1] = build_group(g)
        pend_b = nxt
    for pt in pend_t:
        if pt is not None:
            pt.wait()
    plsc.subcore_barrier()

    for cp in a_cps:
        cp.wait()

    step = _f32(2.0 * HALF / (H - 1))

    def p1(c, pb):
        ty = _f32(-HALF) + (base_row + c).astype(_f32) * step
        w00_p = w00_v.at[pb]
        w01_p = w01_v.at[pb]
        w10_p = w10_v.at[pb]
        w11_p = w11_v.at[pb]
        ri0_p = ri0_v.at[pb]
        ri1_p = ri1_v.at[pb]

        def p1_body(v, _):
            j0 = v * 16
            tx = (j0 + lane).astype(_f32) * step + _f32(-HALF)
            ax = ax_v[pl.ds(c * W + j0, 16)]
            ay = ay_v[pl.ds(c * W + j0, 16)]
            fx = (tx - ax) * SCALE + SHIFT
            fy = (ty - ay) * SCALE + SHIFT
            fx = jnp.clip(fx, -16384.0, 16384.0)
            fy = jnp.clip(fy, -16384.0, 16384.0)
            tix = fx.astype(_i32)
            x0 = tix - jnp.where(fx < tix.astype(_f32), 1, 0)
            tiy = fy.astype(_i32)
            y0 = tiy - jnp.where(fy < tiy.astype(_f32), 1, 0)
            wx1 = fx - x0.astype(_f32)
            wy1 = fy - y0.astype(_f32)
            wx0 = 1.0 - wx1
            wy0 = 1.0 - wy1
            wx0 = wx0 * jnp.where((x0 >= 0) & (x0 < W), 1.0, 0.0)
            wx1 = wx1 * jnp.where((x0 >= -1) & (x0 < W - 1), 1.0, 0.0)
            wy0 = wy0 * jnp.where((y0 >= 0) & (y0 < H), 1.0, 0.0)
            wy1 = wy1 * jnp.where((y0 >= -1) & (y0 < H - 1), 1.0, 0.0)
            neg = x0 < 0
            wx0f = jnp.where(neg, wx1, wx0)
            wx1f = jnp.where(neg, 0.0, wx1)
            w00_p[pl.ds(j0, 16)] = wy0 * wx0f
            w01_p[pl.ds(j0, 16)] = wy0 * wx1f
            w10_p[pl.ds(j0, 16)] = wy1 * wx0f
            w11_p[pl.ds(j0, 16)] = wy1 * wx1f
            xb = jnp.clip(x0, 0, W - 1)
            y0c = jnp.clip(y0, 0, H - 1)
            y1c = jnp.clip(y0 + 1, 0, H - 1)
            tcol = (xb & 1) * 256 + (xb >> 1) + tbase
            ri0_p[pl.ds(j0, 16)] = y0c * W + tcol
            ri1_p[pl.ds(j0, 16)] = y1c * W + tcol
            return _

        lax.fori_loop(0, NVEC, p1_body, None)

    def fire_gathers(pb, sem_g):
        return [
            pltpu.async_copy(table_hbm.at[riv.at[pb, pl.ds(i * 128, 128)]],
                             gv.at[pb, pl.ds(i * 128, 128)], sem_g)
            for riv, gv in ((ri0_v, g0_v), (ri1_v, g1_v))
            for i in range(NDMA)
        ]

    def drain_gathers(sem_g):
        for i in range(2 * NDMA):
            pltpu.make_async_copy(
                table_hbm.at[ri0_v.at[0, pl.ds((i % NDMA) * 128, 128)]],
                g0_v.at[0, pl.ds((i % NDMA) * 128, 128)], sem_g).wait()

    def drain_out():
        for ch in range(C):
            pltpu.make_async_copy(
                outr_v.at[ch], out_hbm.at[0, ch, pl.ds(0, 2)], sem_o).wait()

    lanes0 = [lane * 0 + ch for ch in range(C)]
    lanes1 = [lane * 0 + (ch + 8) for ch in range(C)]

    def p2(pb, cc):
        g0_p = g0_v.at[pb]
        g1_p = g1_v.at[pb]

        def p2_body(v, _):
            j0 = v * 16
            r = j0 + lane
            w00 = w00_v[pb, pl.ds(j0, 16)]
            w01 = w01_v[pb, pl.ds(j0, 16)]
            w10 = w10_v[pb, pl.ds(j0, 16)]
            w11 = w11_v[pb, pl.ds(j0, 16)]
            for ch in range(C):
                v00 = plsc.load_gather(g0_p, [r, lanes0[ch]])
                v01 = plsc.load_gather(g0_p, [r, lanes1[ch]])
                v10 = plsc.load_gather(g1_p, [r, lanes0[ch]])
                v11 = plsc.load_gather(g1_p, [r, lanes1[ch]])
                acc = w00 * v00 + w01 * v01 + w10 * v10 + w11 * v11
                outr_v[ch, cc, pl.ds(j0, 16)] = acc
            return _

        lax.fori_loop(0, NVEC, p2_body, None)

    def fire_out(c0):
        return [
            pltpu.async_copy(outr_v.at[ch],
                             out_hbm.at[0, ch, pl.ds(base_row + c0, 2)],
                             sem_o)
            for ch in range(C)
        ]

    fire_out(0)
    p1(0, 0)
    fire_gathers(0, sem_ga)

    def sample_pair(k, _):
        c1 = 2 * k + 1
        c2 = jnp.minimum(c1 + 1, ROWS_PER_W - 1)
        p1(c1, 1)
        fire_gathers(1, sem_gb)
        drain_gathers(sem_ga)
        drain_out()
        p2(0, 0)
        p1(c2, 0)
        fire_gathers(0, sem_ga)
        drain_gathers(sem_gb)
        p2(1, 1)
        fire_out(2 * k)
        return _

    lax.fori_loop(0, ROWS_PER_W // 2, sample_pair, None)

    drain_gathers(sem_ga)
    drain_out()


def kernel(source_image, alpha):
    img = source_image.reshape(C, NPIX)
    out, _ = _lens_sc(img, alpha.reshape(2 * NPIX))
    return out


# speedup vs baseline: 1.7687x; 1.7687x over previous
import functools

import jax
import jax.numpy as jnp
from jax import lax
from jax.experimental import pallas as pl
from jax.experimental.pallas import tpu as pltpu
from jax.experimental.pallas import tpu_sc as plsc

H = 512
W = 512
C = 8
NPIX = H * W
HALF = 12.8
SCALE = 256.0 / HALF
SHIFT = 255.5
TROWS = H * W

_info = plsc.get_sparse_core_info()
NC, NS = _info.num_cores, _info.num_subcores
NW = NC * NS
ROWS_PER_W = H // NW
NVEC = W // 16
NDMA = W // 128
BY = H // NS

_f32 = jnp.float32
_i32 = jnp.int32


@functools.partial(
    pl.kernel,
    mesh=plsc.VectorSubcoreMesh(core_axis_name="c", subcore_axis_name="s"),
    out_type=(
        jax.ShapeDtypeStruct((1, C, H, W), _f32),
        jax.ShapeDtypeStruct((NC * TROWS, 16), _f32),
    ),
    compiler_params=pltpu.CompilerParams(
        needs_layout_passes=False, use_tc_tiling_on_sc=False),
    scratch_types=[
        pltpu.VMEM((2, C * W + 8), _f32),
        pltpu.VMEM((2, W, 16), _f32),
        pltpu.VMEM((ROWS_PER_W * W,), _f32),
        pltpu.VMEM((ROWS_PER_W * W,), _f32),
        pltpu.VMEM((2, W), _i32),
        pltpu.VMEM((2, W), _i32),
        pltpu.VMEM((2, W), _f32),
        pltpu.VMEM((2, W), _f32),
        pltpu.VMEM((2, W), _f32),
        pltpu.VMEM((2, W), _f32),
        pltpu.VMEM((2, W, 16), _f32),
        pltpu.VMEM((2, W, 16), _f32),
        pltpu.VMEM((C, 2, W), _f32),
        pltpu.SemaphoreType.DMA,
        pltpu.SemaphoreType.DMA,
        pltpu.SemaphoreType.DMA,
        pltpu.SemaphoreType.DMA,
        pltpu.SemaphoreType.DMA,
        pltpu.SemaphoreType.DMA,
        pltpu.SemaphoreType.DMA,
        pltpu.SemaphoreType.DMA,
    ],
)
def _lens_sc(img_hbm, alpha_hbm, out_hbm, table_hbm,
             inb_v, ebo_v, ax_v, ay_v, ri0_v, ri1_v,
             w00_v, w01_v, w10_v, w11_v, g0_v, g1_v, outr_v,
             sem_a, sem_i0, sem_i1, sem_t0, sem_t1, sem_ga, sem_gb, sem_o):
    sc = lax.axis_index("c")
    ss = lax.axis_index("s")
    wid = ss * NC + sc
    base_row = wid * ROWS_PER_W
    lane = lax.iota(_i32, 16)
    ch_pat = lane & 7
    px_pat = lane >> 3
    tbase = sc * TROWS
    sem_i = (sem_i0, sem_i1)
    sem_t = (sem_t0, sem_t1)

    a_cps = [
        pltpu.async_copy(
            alpha_hbm.at[pl.ds(p * NPIX + base_row * W, ROWS_PER_W * W)],
            av, sem_a)
        for p, av in ((0, ax_v), (1, ay_v))
    ]

    y_base = ss * BY

    def fire_build(g):
        return [
            pltpu.async_copy(img_hbm.at[ch, pl.ds((y_base + g) * W, W)],
                             inb_v.at[g & 1, pl.ds(ch * W, W)], sem_i[g & 1])
            for ch in range(C)
        ]

    def build_group(g):
        pb = g & 1
        inb_p = inb_v.at[pb]
        ebo_p = ebo_v.at[pb]

        UB = 8

        def make_body(half):
            def build_rows(t, addr):
                vals = [plsc.load_gather(inb_p, [addr + 2 * i])
                        for i in range(UB)]
                for i in range(UB):
                    ebo_p[half * 256 + t * UB + i, :] = vals[i]
                return addr + 2 * UB
            return build_rows

        lax.fori_loop(0, W // (2 * UB), make_body(0), ch_pat * W + px_pat)
        lax.fori_loop(0, W // (2 * UB), make_body(1),
                      ch_pat * W + px_pat + 1)
        return pltpu.async_copy(
            ebo_p,
            table_hbm.at[pl.ds(tbase + (y_base + g) * W, W)], sem_t[pb])

    pend_b = fire_build(0)
    pend_t = [None, None]
    for g in range(BY):
        nxt = fire_build(g + 1) if g + 1 < BY else None
        for cp in pend_b:
            cp.wait()
        if pend_t[g & 1] is not None:
            pend_t[g & 1].wait()
        pend_t[g & 1] = build_group(g)
        pend_b = nxt
    for pt in pend_t:
        if pt is not None:
            pt.wait()
    plsc.subcore_barrier()

    for cp in a_cps:
        cp.wait()

    step = _f32(2.0 * HALF / (H - 1))

    def p1(c, pb):
        ty = _f32(-HALF) + (base_row + c).astype(_f32) * step
        w00_p = w00_v.at[pb]
        w01_p = w01_v.at[pb]
        w10_p = w10_v.at[pb]
        w11_p = w11_v.at[pb]
        ri0_p = ri0_v.at[pb]
        ri1_p = ri1_v.at[pb]

        def p1_body(v, _):
            j0 = v * 16
            tx = (j0 + lane).astype(_f32) * step + _f32(-HALF)
            ax = ax_v[pl.ds(c * W + j0, 16)]
            ay = ay_v[pl.ds(c * W + j0, 16)]
            fx = (tx - ax) * SCALE + SHIFT
            fy = (ty - ay) * SCALE + SHIFT
            fx = jnp.clip(fx, -16384.0, 16384.0)
            fy = jnp.clip(fy, -16384.0, 16384.0)
            tix = fx.astype(_i32)
            x0 = tix - jnp.where(fx < tix.astype(_f32), 1, 0)
            tiy = fy.astype(_i32)
            y0 = tiy - jnp.where(fy < tiy.astype(_f32), 1, 0)
            wx1 = fx - x0.astype(_f32)
            wy1 = fy - y0.astype(_f32)
            wx0 = 1.0 - wx1
            wy0 = 1.0 - wy1
            wx0 = wx0 * jnp.where((x0 >= 0) & (x0 < W), 1.0, 0.0)
            wx1 = wx1 * jnp.where((x0 >= -1) & (x0 < W - 1), 1.0, 0.0)
            wy0 = wy0 * jnp.where((y0 >= 0) & (y0 < H), 1.0, 0.0)
            wy1 = wy1 * jnp.where((y0 >= -1) & (y0 < H - 1), 1.0, 0.0)
            neg = x0 < 0
            wx0f = jnp.where(neg, wx1, wx0)
            wx1f = jnp.where(neg, 0.0, wx1)
            w00_p[pl.ds(j0, 16)] = wy0 * wx0f
            w01_p[pl.ds(j0, 16)] = wy0 * wx1f
            w10_p[pl.ds(j0, 16)] = wy1 * wx0f
            w11_p[pl.ds(j0, 16)] = wy1 * wx1f
            xb = jnp.clip(x0, 0, W - 1)
            y0c = jnp.clip(y0, 0, H - 1)
            y1c = jnp.clip(y0 + 1, 0, H - 1)
            tcol = (xb & 1) * 256 + (xb >> 1) + tbase
            ri0_p[pl.ds(j0, 16)] = y0c * W + tcol
            ri1_p[pl.ds(j0, 16)] = y1c * W + tcol
            return _

        lax.fori_loop(0, NVEC, p1_body, None)

    def fire_gathers(pb, sem_g):
        return [
            pltpu.async_copy(table_hbm.at[riv.at[pb, pl.ds(i * 128, 128)]],
                             gv.at[pb, pl.ds(i * 128, 128)], sem_g)
            for riv, gv in ((ri0_v, g0_v), (ri1_v, g1_v))
            for i in range(NDMA)
        ]

    def drain_gathers(sem_g):
        for i in range(2 * NDMA):
            pltpu.make_async_copy(
                table_hbm.at[ri0_v.at[0, pl.ds((i % NDMA) * 128, 128)]],
                g0_v.at[0, pl.ds((i % NDMA) * 128, 128)], sem_g).wait()

    def drain_out():
        for ch in range(C):
            pltpu.make_async_copy(
                outr_v.at[ch], out_hbm.at[0, ch, pl.ds(0, 2)], sem_o).wait()

    lanes0 = [lane * 0 + ch for ch in range(C)]
    lanes1 = [lane * 0 + (ch + 8) for ch in range(C)]

    def p2(pb, cc):
        g0_p = g0_v.at[pb]
        g1_p = g1_v.at[pb]

        def p2_body(v, _):
            j0 = v * 16
            r = j0 + lane
            w00 = w00_v[pb, pl.ds(j0, 16)]
            w01 = w01_v[pb, pl.ds(j0, 16)]
            w10 = w10_v[pb, pl.ds(j0, 16)]
            w11 = w11_v[pb, pl.ds(j0, 16)]
            v00 = [plsc.load_gather(g0_p, [r, lanes0[ch]]) for ch in range(C)]
            v01 = [plsc.load_gather(g0_p, [r, lanes1[ch]]) for ch in range(C)]
            v10 = [plsc.load_gather(g1_p, [r, lanes0[ch]]) for ch in range(C)]
            v11 = [plsc.load_gather(g1_p, [r, lanes1[ch]]) for ch in range(C)]
            for ch in range(C):
                acc = ((w00 * v00[ch] + w01 * v01[ch])
                       + (w10 * v10[ch] + w11 * v11[ch]))
                outr_v[ch, cc, pl.ds(j0, 16)] = acc
            return _

        lax.fori_loop(0, NVEC, p2_body, None)

    def fire_out(c0):
        return [
            pltpu.async_copy(outr_v.at[ch],
                             out_hbm.at[0, ch, pl.ds(base_row + c0, 2)],
                             sem_o)
            for ch in range(C)
        ]

    fire_out(0)
    p1(0, 0)
    fire_gathers(0, sem_ga)

    def sample_pair(k, _):
        c1 = 2 * k + 1
        c2 = jnp.minimum(c1 + 1, ROWS_PER_W - 1)
        p1(c1, 1)
        fire_gathers(1, sem_gb)
        drain_gathers(sem_ga)
        drain_out()
        p2(0, 0)
        p1(c2, 0)
        fire_gathers(0, sem_ga)
        drain_gathers(sem_gb)
        p2(1, 1)
        fire_out(2 * k)
        return _

    lax.fori_loop(0, ROWS_PER_W // 2, sample_pair, None)

    drain_gathers(sem_ga)
    drain_out()


def kernel(source_image, alpha):
    img = source_image.reshape(C, NPIX)
    out, _ = _lens_sc(img, alpha.reshape(2 * NPIX))
    return out


# speedup vs baseline: 1.7736x; 1.0028x over previous
import functools

import jax
import jax.numpy as jnp
from jax import lax
from jax.experimental import pallas as pl
from jax.experimental.pallas import tpu as pltpu
from jax.experimental.pallas import tpu_sc as plsc

H = 512
W = 512
C = 8
NPIX = H * W
HALF = 12.8
SCALE = 256.0 / HALF
SHIFT = 255.5
TROWS = H * W

_info = plsc.get_sparse_core_info()
NC, NS = _info.num_cores, _info.num_subcores
NW = NC * NS
ROWS_PER_W = H // NW
NVEC = W // 16
NDMA = W // 128
BY = H // NS

_f32 = jnp.float32
_i32 = jnp.int32


@functools.partial(
    pl.kernel,
    mesh=plsc.VectorSubcoreMesh(core_axis_name="c", subcore_axis_name="s"),
    out_type=(
        jax.ShapeDtypeStruct((1, C, H, W), _f32),
        jax.ShapeDtypeStruct((NC * TROWS, 16), _f32),
    ),
    compiler_params=pltpu.CompilerParams(
        needs_layout_passes=False, use_tc_tiling_on_sc=False),
    scratch_types=[
        pltpu.VMEM((2, C * W + 8), _f32),
        pltpu.VMEM((2, W, 16), _f32),
        pltpu.VMEM((ROWS_PER_W, W), _f32),
        pltpu.VMEM((ROWS_PER_W, W), _f32),
        pltpu.VMEM((W,), _f32),
        pltpu.VMEM((2, W), _i32),
        pltpu.VMEM((2, W), _i32),
        pltpu.VMEM((2, W), _f32),
        pltpu.VMEM((2, W), _f32),
        pltpu.VMEM((2, W), _f32),
        pltpu.VMEM((2, W), _f32),
        pltpu.VMEM((2, W, 16), _f32),
        pltpu.VMEM((2, W, 16), _f32),
        pltpu.VMEM((C, 2, W), _f32),
        pltpu.SemaphoreType.DMA,
        pltpu.SemaphoreType.DMA,
        pltpu.SemaphoreType.DMA,
        pltpu.SemaphoreType.DMA,
        pltpu.SemaphoreType.DMA,
        pltpu.SemaphoreType.DMA,
        pltpu.SemaphoreType.DMA,
        pltpu.SemaphoreType.DMA,
    ],
)
def _lens_sc(img_hbm, alpha_hbm, out_hbm, table_hbm,
             inb_v, ebo_v, ax_v, ay_v, tx_v, ri0_v, ri1_v,
             w00_v, w01_v, w10_v, w11_v, g0_v, g1_v, outr_v,
             sem_a, sem_i0, sem_i1, sem_t0, sem_t1, sem_ga, sem_gb, sem_o):
    sc = lax.axis_index("c")
    ss = lax.axis_index("s")
    wid = ss * NC + sc
    base_row = wid * ROWS_PER_W
    lane = lax.iota(_i32, 16)
    ch_pat = lane & 7
    px_pat = lane >> 3
    tbase = sc * TROWS
    sem_i = (sem_i0, sem_i1)
    sem_t = (sem_t0, sem_t1)

    a_cps = [
        pltpu.async_copy(
            alpha_hbm.at[p, pl.ds(base_row, ROWS_PER_W)], av, sem_a)
        for p, av in ((0, ax_v), (1, ay_v))
    ]

    y_base = ss * BY

    def fire_build(g):
        return [
            pltpu.async_copy(img_hbm.at[0, ch, y_base + g],
                             inb_v.at[g & 1, pl.ds(ch * W, W)], sem_i[g & 1])
            for ch in range(C)
        ]

    def build_group(g):
        pb = g & 1
        inb_p = inb_v.at[pb]
        ebo_p = ebo_v.at[pb]

        UB = 8

        def make_body(half):
            def build_rows(t, addr):
                vals = [plsc.load_gather(inb_p, [addr + 2 * i])
                        for i in range(UB)]
                for i in range(UB):
                    ebo_p[half * 256 + t * UB + i, :] = vals[i]
                return addr + 2 * UB
            return build_rows

        lax.fori_loop(0, W // (2 * UB), make_body(0), ch_pat * W + px_pat)
        lax.fori_loop(0, W // (2 * UB), make_body(1),
                      ch_pat * W + px_pat + 1)
        return pltpu.async_copy(
            ebo_p,
            table_hbm.at[pl.ds(tbase + (y_base + g) * W, W)], sem_t[pb])

    pend_b = fire_build(0)
    pend_t = [None, None]
    for g in range(BY):
        nxt = fire_build(g + 1) if g + 1 < BY else None
        for cp in pend_b:
            cp.wait()
        if pend_t[g & 1] is not None:
            pend_t[g & 1].wait()
        pend_t[g & 1] = build_group(g)
        pend_b = nxt
    for pt in pend_t:
        if pt is not None:
            pt.wait()
    plsc.subcore_barrier()

    for cp in a_cps:
        cp.wait()

    step = _f32(2.0 * HALF / (H - 1))

    def tx_init(v, _):
        j0 = v * 16
        tx_v[pl.ds(j0, 16)] = ((j0 + lane).astype(_f32) * step
                               + _f32(-HALF)) * SCALE + SHIFT
        return _

    lax.fori_loop(0, NVEC, tx_init, None)

    def p1(c, pb):
        ty = _f32(-HALF) + (base_row + c).astype(_f32) * step
        w00_p = w00_v.at[pb]
        w01_p = w01_v.at[pb]
        w10_p = w10_v.at[pb]
        w11_p = w11_v.at[pb]
        ri0_p = ri0_v.at[pb]
        ri1_p = ri1_v.at[pb]

        def p1_body(v, _):
            j0 = v * 16
            ax = ax_v[c, pl.ds(j0, 16)]
            ay = ay_v[c, pl.ds(j0, 16)]
            fx = tx_v[pl.ds(j0, 16)] - ax * SCALE
            fy = (ty - ay) * SCALE + SHIFT
            fx = jnp.clip(fx, -16384.0, 16384.0)
            fy = jnp.clip(fy, -16384.0, 16384.0)
            tix = fx.astype(_i32)
            x0 = tix - jnp.where(fx < tix.astype(_f32), 1, 0)
            tiy = fy.astype(_i32)
            y0 = tiy - jnp.where(fy < tiy.astype(_f32), 1, 0)
            wx1 = fx - x0.astype(_f32)
            wy1 = fy - y0.astype(_f32)
            wx0 = 1.0 - wx1
            wy0 = 1.0 - wy1
            x0u = x0.astype(jnp.uint32)
            y0u = y0.astype(jnp.uint32)
            wx0 = wx0 * jnp.where(x0u < W, 1.0, 0.0)
            wx1 = wx1 * jnp.where(x0u + 1 < W, 1.0, 0.0)
            wy0 = wy0 * jnp.where(y0u < H, 1.0, 0.0)
            wy1 = wy1 * jnp.where(y0u + 1 < H, 1.0, 0.0)
            neg = x0 < 0
            wx0f = jnp.where(neg, wx1, wx0)
            wx1f = jnp.where(neg, 0.0, wx1)
            w00_p[pl.ds(j0, 16)] = wy0 * wx0f
            w01_p[pl.ds(j0, 16)] = wy0 * wx1f
            w10_p[pl.ds(j0, 16)] = wy1 * wx0f
            w11_p[pl.ds(j0, 16)] = wy1 * wx1f
            xb = jnp.clip(x0, 0, W - 1)
            y0c = jnp.clip(y0, 0, H - 1)
            y1c = jnp.clip(y0 + 1, 0, H - 1)
            tcol = (xb & 1) * 256 + (xb >> 1) + tbase
            ri0_p[pl.ds(j0, 16)] = y0c * W + tcol
            ri1_p[pl.ds(j0, 16)] = y1c * W + tcol
            return _

        lax.fori_loop(0, NVEC, p1_body, None)

    def fire_gathers(pb, sem_g):
        return [
            pltpu.async_copy(table_hbm.at[riv.at[pb, pl.ds(i * 128, 128)]],
                             gv.at[pb, pl.ds(i * 128, 128)], sem_g)
            for riv, gv in ((ri0_v, g0_v), (ri1_v, g1_v))
            for i in range(NDMA)
        ]

    def drain_gathers(sem_g):
        for i in range(2 * NDMA):
            pltpu.make_async_copy(
                table_hbm.at[ri0_v.at[0, pl.ds((i % NDMA) * 128, 128)]],
                g0_v.at[0, pl.ds((i % NDMA) * 128, 128)], sem_g).wait()

    def drain_out():
        for ch in range(C):
            pltpu.make_async_copy(
                outr_v.at[ch], out_hbm.at[0, ch, pl.ds(0, 2)], sem_o).wait()

    lanes0 = [lane * 0 + ch for ch in range(C)]
    lanes1 = [lane * 0 + (ch + 8) for ch in range(C)]

    def p2(pb, cc):
        g0_p = g0_v.at[pb]
        g1_p = g1_v.at[pb]

        def p2_body(v, _):
            j0 = v * 16
            r = j0 + lane
            w00 = w00_v[pb, pl.ds(j0, 16)]
            w01 = w01_v[pb, pl.ds(j0, 16)]
            w10 = w10_v[pb, pl.ds(j0, 16)]
            w11 = w11_v[pb, pl.ds(j0, 16)]
            v00 = [plsc.load_gather(g0_p, [r, lanes0[ch]]) for ch in range(C)]
            v01 = [plsc.load_gather(g0_p, [r, lanes1[ch]]) for ch in range(C)]
            v10 = [plsc.load_gather(g1_p, [r, lanes0[ch]]) for ch in range(C)]
            v11 = [plsc.load_gather(g1_p, [r, lanes1[ch]]) for ch in range(C)]
            for ch in range(C):
                acc = ((w00 * v00[ch] + w01 * v01[ch])
                       + (w10 * v10[ch] + w11 * v11[ch]))
                outr_v[ch, cc, pl.ds(j0, 16)] = acc
            return _

        lax.fori_loop(0, NVEC, p2_body, None)

    def fire_out(c0):
        return [
            pltpu.async_copy(outr_v.at[ch],
                             out_hbm.at[0, ch, pl.ds(base_row + c0, 2)],
                             sem_o)
            for ch in range(C)
        ]

    fire_out(0)
    p1(0, 0)
    fire_gathers(0, sem_ga)

    def sample_pair(k, _):
        c1 = 2 * k + 1
        c2 = jnp.minimum(c1 + 1, ROWS_PER_W - 1)
        p1(c1, 1)
        fire_gathers(1, sem_gb)
        drain_gathers(sem_ga)
        drain_out()
        p2(0, 0)
        p1(c2, 0)
        fire_gathers(0, sem_ga)
        drain_gathers(sem_gb)
        p2(1, 1)
        fire_out(2 * k)
        return _

    lax.fori_loop(0, ROWS_PER_W // 2, sample_pair, None)

    drain_gathers(sem_ga)
    drain_out()


def kernel(source_image, alpha):
    out, _ = _lens_sc(source_image, alpha)
    return out


# speedup vs baseline: 1.7825x; 1.0050x over previous
import functools

import jax
import jax.numpy as jnp
from jax import lax
from jax.experimental import pallas as pl
from jax.experimental.pallas import tpu as pltpu
from jax.experimental.pallas import tpu_sc as plsc

H = 512
W = 512
C = 8
NPIX = H * W
HALF = 12.8
SCALE = 256.0 / HALF
SHIFT = 255.5
TROWS = H * W

_info = plsc.get_sparse_core_info()
NC, NS = _info.num_cores, _info.num_subcores
NW = NC * NS
ROWS_PER_W = H // NW
NVEC = W // 16
NDMA = W // 128
BY = H // NS

_f32 = jnp.float32
_i32 = jnp.int32


@functools.partial(
    pl.kernel,
    mesh=plsc.VectorSubcoreMesh(core_axis_name="c", subcore_axis_name="s"),
    out_type=(
        jax.ShapeDtypeStruct((1, C, H, W), _f32),
        jax.ShapeDtypeStruct((NC * TROWS, 16), _f32),
    ),
    compiler_params=pltpu.CompilerParams(
        needs_layout_passes=False, use_tc_tiling_on_sc=False),
    scratch_types=[
        pltpu.VMEM((2, C * W + 8), _f32),
        pltpu.VMEM((2, W, 16), _f32),
        pltpu.VMEM((ROWS_PER_W, W), _f32),
        pltpu.VMEM((ROWS_PER_W, W), _f32),
        pltpu.VMEM((W,), _f32),
        pltpu.VMEM((2, W), _i32),
        pltpu.VMEM((2, W), _i32),
        pltpu.VMEM((2, W), _f32),
        pltpu.VMEM((2, W), _f32),
        pltpu.VMEM((2, W), _f32),
        pltpu.VMEM((2, W), _f32),
        pltpu.VMEM((2, W, 16), _f32),
        pltpu.VMEM((2, W, 16), _f32),
        pltpu.VMEM((C, 2, W), _f32),
        pltpu.SemaphoreType.DMA,
        pltpu.SemaphoreType.DMA,
        pltpu.SemaphoreType.DMA,
        pltpu.SemaphoreType.DMA,
        pltpu.SemaphoreType.DMA,
        pltpu.SemaphoreType.DMA,
        pltpu.SemaphoreType.DMA,
        pltpu.SemaphoreType.DMA,
    ],
)
def _lens_sc(img_hbm, alpha_hbm, out_hbm, table_hbm,
             inb_v, ebo_v, ax_v, ay_v, tx_v, ri0_v, ri1_v,
             w00_v, w01_v, w10_v, w11_v, g0_v, g1_v, outr_v,
             sem_a, sem_i0, sem_i1, sem_t0, sem_t1, sem_ga, sem_gb, sem_o):
    sc = lax.axis_index("c")
    ss = lax.axis_index("s")
    wid = ss * NC + sc
    base_row = wid * ROWS_PER_W
    lane = lax.iota(_i32, 16)
    ch_pat = lane & 7
    px_pat = lane >> 3
    tbase = sc * TROWS
    sem_i = (sem_i0, sem_i1)
    sem_t = (sem_t0, sem_t1)

    a_cps = [
        pltpu.async_copy(
            alpha_hbm.at[p, pl.ds(base_row, ROWS_PER_W)], av, sem_a)
        for p, av in ((0, ax_v), (1, ay_v))
    ]

    y_base = ss * BY

    def fire_build(g):
        return [
            pltpu.async_copy(img_hbm.at[0, ch, y_base + g],
                             inb_v.at[g & 1, pl.ds(ch * W, W)], sem_i[g & 1])
            for ch in range(C)
        ]

    def build_group(g):
        pb = g & 1
        inb_p = inb_v.at[pb]
        ebo_p = ebo_v.at[pb]

        UB = 8

        def make_body(half):
            def build_rows(t, addr):
                vals = [plsc.load_gather(inb_p, [addr + 2 * i])
                        for i in range(UB)]
                for i in range(UB):
                    ebo_p[half * 256 + t * UB + i, :] = vals[i]
                return addr + 2 * UB
            return build_rows

        lax.fori_loop(0, W // (2 * UB), make_body(0), ch_pat * W + px_pat)
        lax.fori_loop(0, W // (2 * UB), make_body(1),
                      ch_pat * W + px_pat + 1)
        return pltpu.async_copy(
            ebo_p,
            table_hbm.at[pl.ds(tbase + (y_base + g) * W, W)], sem_t[pb])

    pend_b = fire_build(0)
    pend_t = [None, None]
    for g in range(BY):
        nxt = fire_build(g + 1) if g + 1 < BY else None
        for cp in pend_b:
            cp.wait()
        if pend_t[g & 1] is not None:
            pend_t[g & 1].wait()
        pend_t[g & 1] = build_group(g)
        pend_b = nxt
    for pt in pend_t:
        if pt is not None:
            pt.wait()
    plsc.subcore_barrier()

    for cp in a_cps:
        cp.wait()

    step = _f32(2.0 * HALF / (H - 1))

    def tx_init(v, _):
        j0 = v * 16
        tx_v[pl.ds(j0, 16)] = ((j0 + lane).astype(_f32) * step
                               + _f32(-HALF)) * SCALE + SHIFT
        return _

    lax.fori_loop(0, NVEC, tx_init, None)

    def p1(c, pb):
        ty = _f32(-HALF) + (base_row + c).astype(_f32) * step
        w00_p = w00_v.at[pb]
        w01_p = w01_v.at[pb]
        w10_p = w10_v.at[pb]
        w11_p = w11_v.at[pb]
        ri0_p = ri0_v.at[pb]
        ri1_p = ri1_v.at[pb]

        def p1_body(v, _):
            j0 = v * 16
            ax = ax_v[c, pl.ds(j0, 16)]
            ay = ay_v[c, pl.ds(j0, 16)]
            fx = tx_v[pl.ds(j0, 16)] - ax * SCALE
            fy = (ty - ay) * SCALE + SHIFT
            fx = jnp.clip(fx, -16384.0, 16384.0)
            fy = jnp.clip(fy, -16384.0, 16384.0)
            tix = fx.astype(_i32)
            x0 = tix - jnp.where(fx < tix.astype(_f32), 1, 0)
            tiy = fy.astype(_i32)
            y0 = tiy - jnp.where(fy < tiy.astype(_f32), 1, 0)
            wx1 = fx - x0.astype(_f32)
            wy1 = fy - y0.astype(_f32)
            wx0 = 1.0 - wx1
            wy0 = 1.0 - wy1
            x0u = x0.astype(jnp.uint32)
            y0u = y0.astype(jnp.uint32)
            wx0 = wx0 * jnp.where(x0u < W, 1.0, 0.0)
            wx1 = wx1 * jnp.where(x0u + 1 < W, 1.0, 0.0)
            wy0 = wy0 * jnp.where(y0u < H, 1.0, 0.0)
            wy1 = wy1 * jnp.where(y0u + 1 < H, 1.0, 0.0)
            neg = x0 < 0
            wx0f = jnp.where(neg, wx1, wx0)
            wx1f = jnp.where(neg, 0.0, wx1)
            w00_p[pl.ds(j0, 16)] = wy0 * wx0f
            w01_p[pl.ds(j0, 16)] = wy0 * wx1f
            w10_p[pl.ds(j0, 16)] = wy1 * wx0f
            w11_p[pl.ds(j0, 16)] = wy1 * wx1f
            xb = jnp.clip(x0, 0, W - 1)
            y0c = jnp.clip(y0, 0, H - 1)
            y1c = jnp.clip(y0 + 1, 0, H - 1)
            tcol = (xb & 1) * 256 + (xb >> 1) + tbase
            ri0_p[pl.ds(j0, 16)] = y0c * W + tcol
            ri1_p[pl.ds(j0, 16)] = y1c * W + tcol
            return _

        lax.fori_loop(0, NVEC, p1_body, None)

    def fire_gathers(pb, sem_g):
        return [
            pltpu.async_copy(table_hbm.at[riv.at[pb]], gv.at[pb], sem_g)
            for riv, gv in ((ri0_v, g0_v), (ri1_v, g1_v))
        ]

    def drain_gathers(sem_g):
        for gv in (g0_v, g1_v):
            pltpu.make_async_copy(
                table_hbm.at[ri0_v.at[0]], gv.at[0], sem_g).wait()

    def drain_out():
        pltpu.make_async_copy(
            out_hbm.at[0, :, pl.ds(0, 2)], outr_v, sem_o).wait()

    lanes0 = [lane * 0 + ch for ch in range(C)]
    lanes1 = [lane * 0 + (ch + 8) for ch in range(C)]

    def p2(pb, cc):
        g0_p = g0_v.at[pb]
        g1_p = g1_v.at[pb]

        def p2_body(v, _):
            j0 = v * 16
            r = j0 + lane
            w00 = w00_v[pb, pl.ds(j0, 16)]
            w01 = w01_v[pb, pl.ds(j0, 16)]
            w10 = w10_v[pb, pl.ds(j0, 16)]
            w11 = w11_v[pb, pl.ds(j0, 16)]
            v00 = [plsc.load_gather(g0_p, [r, lanes0[ch]]) for ch in range(C)]
            v01 = [plsc.load_gather(g0_p, [r, lanes1[ch]]) for ch in range(C)]
            v10 = [plsc.load_gather(g1_p, [r, lanes0[ch]]) for ch in range(C)]
            v11 = [plsc.load_gather(g1_p, [r, lanes1[ch]]) for ch in range(C)]
            for ch in range(C):
                acc = ((w00 * v00[ch] + w01 * v01[ch])
                       + (w10 * v10[ch] + w11 * v11[ch]))
                outr_v[ch, cc, pl.ds(j0, 16)] = acc
            return _

        lax.fori_loop(0, NVEC, p2_body, None)

    def fire_out(c0):
        return [
            pltpu.async_copy(outr_v.at[ch],
                             out_hbm.at[0, ch, pl.ds(base_row + c0, 2)],
                             sem_o)
            for ch in range(C)
        ]

    fire_out(0)
    p1(0, 0)
    fire_gathers(0, sem_ga)

    def sample_pair(k, _):
        c1 = 2 * k + 1
        c2 = jnp.minimum(c1 + 1, ROWS_PER_W - 1)
        p1(c1, 1)
        fire_gathers(1, sem_gb)
        drain_gathers(sem_ga)
        drain_out()
        p2(0, 0)
        p1(c2, 0)
        fire_gathers(0, sem_ga)
        drain_gathers(sem_gb)
        p2(1, 1)
        fire_out(2 * k)
        return _

    lax.fori_loop(0, ROWS_PER_W // 2, sample_pair, None)

    drain_gathers(sem_ga)
    drain_out()


def kernel(source_image, alpha):
    out, _ = _lens_sc(source_image, alpha)
    return out


# speedup vs baseline: 1.7837x; 1.0007x over previous
import functools

import jax
import jax.numpy as jnp
from jax import lax
from jax.experimental import pallas as pl
from jax.experimental.pallas import tpu as pltpu
from jax.experimental.pallas import tpu_sc as plsc

H = 512
W = 512
C = 8
NPIX = H * W
HALF = 12.8
SCALE = 256.0 / HALF
SHIFT = 255.5
TROWS = H * W

_info = plsc.get_sparse_core_info()
NC, NS = _info.num_cores, _info.num_subcores
NW = NC * NS
ROWS_PER_W = H // NW
NVEC = W // 16
NDMA = W // 128
BY = H // NS

_f32 = jnp.float32
_i32 = jnp.int32


@functools.partial(
    pl.kernel,
    mesh=plsc.VectorSubcoreMesh(core_axis_name="c", subcore_axis_name="s"),
    out_type=jax.ShapeDtypeStruct((1, C, H, W), _f32),
    compiler_params=pltpu.CompilerParams(
        needs_layout_passes=False, use_tc_tiling_on_sc=False),
    scratch_types=[
        pltpu.HBM((NC * TROWS, 16), _f32),
        pltpu.VMEM((2, C * W + 8), _f32),
        pltpu.VMEM((2, W, 16), _f32),
        pltpu.VMEM((ROWS_PER_W, W), _f32),
        pltpu.VMEM((ROWS_PER_W, W), _f32),
        pltpu.VMEM((W,), _f32),
        pltpu.VMEM((2, W), _i32),
        pltpu.VMEM((2, W), _i32),
        pltpu.VMEM((2, W), _f32),
        pltpu.VMEM((2, W), _f32),
        pltpu.VMEM((2, W), _f32),
        pltpu.VMEM((2, W), _f32),
        pltpu.VMEM((2, W, 16), _f32),
        pltpu.VMEM((2, W, 16), _f32),
        pltpu.VMEM((C, 2, W), _f32),
        pltpu.SemaphoreType.DMA,
        pltpu.SemaphoreType.DMA,
        pltpu.SemaphoreType.DMA,
        pltpu.SemaphoreType.DMA,
        pltpu.SemaphoreType.DMA,
        pltpu.SemaphoreType.DMA,
        pltpu.SemaphoreType.DMA,
        pltpu.SemaphoreType.DMA,
    ],
)
def _lens_sc(img_hbm, alpha_hbm, out_hbm, table_hbm,
             inb_v, ebo_v, ax_v, ay_v, tx_v, ri0_v, ri1_v,
             w00_v, w01_v, w10_v, w11_v, g0_v, g1_v, outr_v,
             sem_a, sem_i0, sem_i1, sem_t0, sem_t1, sem_ga, sem_gb, sem_o):
    sc = lax.axis_index("c")
    ss = lax.axis_index("s")
    wid = ss * NC + sc
    base_row = wid * ROWS_PER_W
    lane = lax.iota(_i32, 16)
    ch_pat = lane & 7
    px_pat = lane >> 3
    tbase = sc * TROWS
    sem_i = (sem_i0, sem_i1)
    sem_t = (sem_t0, sem_t1)

    a_cps = [
        pltpu.async_copy(
            alpha_hbm.at[p, pl.ds(base_row, ROWS_PER_W)], av, sem_a)
        for p, av in ((0, ax_v), (1, ay_v))
    ]

    y_base = ss * BY

    def fire_build(g):
        return [
            pltpu.async_copy(img_hbm.at[0, ch, y_base + g],
                             inb_v.at[g & 1, pl.ds(ch * W, W)], sem_i[g & 1])
            for ch in range(C)
        ]

    def build_group(g):
        pb = g & 1
        inb_p = inb_v.at[pb]
        ebo_p = ebo_v.at[pb]

        UB = 8

        def make_body(half):
            def build_rows(t, addr):
                vals = [plsc.load_gather(inb_p, [addr + 2 * i])
                        for i in range(UB)]
                for i in range(UB):
                    ebo_p[half * 256 + t * UB + i, :] = vals[i]
                return addr + 2 * UB
            return build_rows

        lax.fori_loop(0, W // (2 * UB), make_body(0), ch_pat * W + px_pat)
        lax.fori_loop(0, W // (2 * UB), make_body(1),
                      ch_pat * W + px_pat + 1)
        return pltpu.async_copy(
            ebo_p,
            table_hbm.at[pl.ds(tbase + (y_base + g) * W, W)], sem_t[pb])

    pend_b = fire_build(0)
    pend_t = [None, None]
    for g in range(BY):
        nxt = fire_build(g + 1) if g + 1 < BY else None
        for cp in pend_b:
            cp.wait()
        if pend_t[g & 1] is not None:
            pend_t[g & 1].wait()
        pend_t[g & 1] = build_group(g)
        pend_b = nxt
    for pt in pend_t:
        if pt is not None:
            pt.wait()
    plsc.subcore_barrier()

    for cp in a_cps:
        cp.wait()

    step = _f32(2.0 * HALF / (H - 1))

    def tx_init(v, _):
        j0 = v * 16
        tx_v[pl.ds(j0, 16)] = ((j0 + lane).astype(_f32) * step
                               + _f32(-HALF)) * SCALE + SHIFT
        return _

    lax.fori_loop(0, NVEC, tx_init, None)

    def p1(c, pb):
        ty = _f32(-HALF) + (base_row + c).astype(_f32) * step
        w00_p = w00_v.at[pb]
        w01_p = w01_v.at[pb]
        w10_p = w10_v.at[pb]
        w11_p = w11_v.at[pb]
        ri0_p = ri0_v.at[pb]
        ri1_p = ri1_v.at[pb]

        def p1_body(v, _):
            j0 = v * 16
            ax = ax_v[c, pl.ds(j0, 16)]
            ay = ay_v[c, pl.ds(j0, 16)]
            fx = tx_v[pl.ds(j0, 16)] - ax * SCALE
            fy = (ty - ay) * SCALE + SHIFT
            fx = jnp.clip(fx, -16384.0, 16384.0)
            fy = jnp.clip(fy, -16384.0, 16384.0)
            tix = fx.astype(_i32)
            x0 = tix - jnp.where(fx < tix.astype(_f32), 1, 0)
            tiy = fy.astype(_i32)
            y0 = tiy - jnp.where(fy < tiy.astype(_f32), 1, 0)
            wx1 = fx - x0.astype(_f32)
            wy1 = fy - y0.astype(_f32)
            wx0 = 1.0 - wx1
            wy0 = 1.0 - wy1
            x0u = x0.astype(jnp.uint32)
            y0u = y0.astype(jnp.uint32)
            wx0 = wx0 * jnp.where(x0u < W, 1.0, 0.0)
            wx1 = wx1 * jnp.where(x0u + 1 < W, 1.0, 0.0)
            wy0 = wy0 * jnp.where(y0u < H, 1.0, 0.0)
            wy1 = wy1 * jnp.where(y0u + 1 < H, 1.0, 0.0)
            neg = x0 < 0
            wx0f = jnp.where(neg, wx1, wx0)
            wx1f = jnp.where(neg, 0.0, wx1)
            w00_p[pl.ds(j0, 16)] = wy0 * wx0f
            w01_p[pl.ds(j0, 16)] = wy0 * wx1f
            w10_p[pl.ds(j0, 16)] = wy1 * wx0f
            w11_p[pl.ds(j0, 16)] = wy1 * wx1f
            xb = jnp.clip(x0, 0, W - 1)
            y0c = jnp.clip(y0, 0, H - 1)
            y1c = jnp.clip(y0 + 1, 0, H - 1)
            tcol = (xb & 1) * 256 + (xb >> 1) + tbase
            ri0_p[pl.ds(j0, 16)] = y0c * W + tcol
            ri1_p[pl.ds(j0, 16)] = y1c * W + tcol
            return _

        lax.fori_loop(0, NVEC, p1_body, None)

    def fire_gathers(pb, sem_g):
        return [
            pltpu.async_copy(table_hbm.at[riv.at[pb]], gv.at[pb], sem_g)
            for riv, gv in ((ri0_v, g0_v), (ri1_v, g1_v))
        ]

    def drain_gathers(sem_g):
        for gv in (g0_v, g1_v):
            pltpu.make_async_copy(
                table_hbm.at[ri0_v.at[0]], gv.at[0], sem_g).wait()

    def drain_out():
        pltpu.make_async_copy(
            out_hbm.at[0, :, pl.ds(0, 2)], outr_v, sem_o).wait()

    lanes0 = [lane * 0 + ch for ch in range(C)]
    lanes1 = [lane * 0 + (ch + 8) for ch in range(C)]

    def p2(pb, cc):
        g0_p = g0_v.at[pb]
        g1_p = g1_v.at[pb]

        def p2_body(v, _):
            j0 = v * 16
            r = j0 + lane
            w00 = w00_v[pb, pl.ds(j0, 16)]
            w01 = w01_v[pb, pl.ds(j0, 16)]
            w10 = w10_v[pb, pl.ds(j0, 16)]
            w11 = w11_v[pb, pl.ds(j0, 16)]
            v00 = [plsc.load_gather(g0_p, [r, lanes0[ch]]) for ch in range(C)]
            v01 = [plsc.load_gather(g0_p, [r, lanes1[ch]]) for ch in range(C)]
            v10 = [plsc.load_gather(g1_p, [r, lanes0[ch]]) for ch in range(C)]
            v11 = [plsc.load_gather(g1_p, [r, lanes1[ch]]) for ch in range(C)]
            for ch in range(C):
                acc = ((w00 * v00[ch] + w01 * v01[ch])
                       + (w10 * v10[ch] + w11 * v11[ch]))
                outr_v[ch, cc, pl.ds(j0, 16)] = acc
            return _

        lax.fori_loop(0, NVEC, p2_body, None)

    def fire_out(c0):
        return [
            pltpu.async_copy(outr_v.at[ch],
                             out_hbm.at[0, ch, pl.ds(base_row + c0, 2)],
                             sem_o)
            for ch in range(C)
        ]

    fire_out(0)
    p1(0, 0)
    fire_gathers(0, sem_ga)

    def sample_pair(k, _):
        c1 = 2 * k + 1
        c2 = jnp.minimum(c1 + 1, ROWS_PER_W - 1)
        p1(c1, 1)
        fire_gathers(1, sem_gb)
        drain_gathers(sem_ga)
        drain_out()
        p2(0, 0)
        p1(c2, 0)
        fire_gathers(0, sem_ga)
        drain_gathers(sem_gb)
        p2(1, 1)
        fire_out(2 * k)
        return _

    lax.fori_loop(0, ROWS_PER_W // 2, sample_pair, None)

    drain_gathers(sem_ga)
    drain_out()


def kernel(source_image, alpha):
    return _lens_sc(source_image, alpha)
